# baseline (speedup 1.0000x reference)
import jax
import jax.numpy as jnp
from jax.experimental import pallas as pl
from jax.experimental.pallas import tpu as pltpu

N = 8192
D = 64
BM = 256


def _mm_block(a_ref, xt_ref, ot_ref):
    ot_ref[...] = jax.lax.dot_general(
        xt_ref[...], a_ref[...],
        dimension_numbers=(((1,), (1,)), ((), ())),
        preferred_element_type=jnp.float32)


def kernel(x, A_hat):
    xt = x.T
    out_t = pl.pallas_call(
        _mm_block,
        grid=(N // BM,),
        in_specs=[
            pl.BlockSpec((BM, N), lambda i: (i, 0)),
            pl.BlockSpec((D, N), lambda i: (0, 0)),
        ],
        out_specs=pl.BlockSpec((D, BM), lambda i: (0, i)),
        out_shape=jax.ShapeDtypeStruct((D, N), jnp.float32),
        compiler_params=pltpu.CompilerParams(
            dimension_semantics=("parallel",),
        ),
    )(A_hat, xt)
    return out_t.T


# speedup vs baseline: 1.0046x; 1.0046x over previous
import jax
import jax.numpy as jnp
from jax.experimental import pallas as pl
from jax.experimental.pallas import tpu as pltpu

N = 8192
D = 64
BM = 512


def _mm_block(a_ref, xt_ref, ot_ref):
    ot_ref[...] = jax.lax.dot_general(
        xt_ref[...], a_ref[...],
        dimension_numbers=(((1,), (1,)), ((), ())),
        preferred_element_type=jnp.float32)


def kernel(x, A_hat):
    xt = x.T
    out_t = pl.pallas_call(
        _mm_block,
        grid=(N // BM,),
        in_specs=[
            pl.BlockSpec((BM, N), lambda i: (i, 0)),
            pl.BlockSpec((D, N), lambda i: (0, 0)),
        ],
        out_specs=pl.BlockSpec((D, BM), lambda i: (0, i)),
        out_shape=jax.ShapeDtypeStruct((D, N), jnp.float32),
        compiler_params=pltpu.CompilerParams(
            dimension_semantics=("parallel",),
        ),
    )(A_hat, xt)
    return out_t.T


# speedup vs baseline: 1.0083x; 1.0037x over previous
import jax
import jax.numpy as jnp
from jax.experimental import pallas as pl
from jax.experimental.pallas import tpu as pltpu

N = 8192
D = 64
BM = 512


def _mm_block(a_ref, xt_ref, ot_ref):
    ot_ref[...] = jax.lax.dot_general(
        xt_ref[...], a_ref[...],
        dimension_numbers=(((1,), (1,)), ((), ())),
        preferred_element_type=jnp.float32)


def kernel(x, A_hat):
    xt = x.T
    out_t = pl.pallas_call(
        _mm_block,
        grid=(N // BM,),
        in_specs=[
            pl.BlockSpec((BM, N), lambda i: (i, 0)),
            pl.BlockSpec((D, N), lambda i: (0, 0)),
        ],
        out_specs=pl.BlockSpec((D, BM), lambda i: (0, i)),
        out_shape=jax.ShapeDtypeStruct((D, N), jnp.float32),
        compiler_params=pltpu.CompilerParams(
            dimension_semantics=("arbitrary",),
        ),
    )(A_hat, xt)
    return out_t.T
